# baseline (speedup 1.0000x reference)
import functools

import jax
import jax.numpy as jnp
from jax import lax
from jax.experimental import pallas as pl
from jax.experimental.pallas import tpu as pltpu
from jax.experimental.pallas import tpu_sc as plsc

LN_EPS = 1e-5
LANES = 16
NC = 2
NS = 16
NW = NC * NS
KCODES = 8
D = 128
SUB = 16
OUTER = 128
NSUB = OUTER // SUB


def _proj_body(cb_ref, w_ref, b_ref, o_ref):
    o_ref[...] = (
        jnp.dot(cb_ref[...], w_ref[...], preferred_element_type=jnp.float32)
        + b_ref[...]
    ) * 0.125


def _ln_body(x_ref, g_ref, bb_ref, o_ref):
    x = x_ref[...]
    mu = jnp.mean(x, axis=-1, keepdims=True)
    xc = x - mu
    var = jnp.mean(xc * xc, axis=-1, keepdims=True)
    o_ref[...] = xc * lax.rsqrt(var + LN_EPS) * g_ref[...] + bb_ref[...]


@functools.lru_cache(maxsize=None)
def _make_sc_gather(T):
    TPW = T // NW
    NOUT = TPW // OUTER
    mesh = plsc.VectorSubcoreMesh(core_axis_name="c", subcore_axis_name="s")

    @functools.partial(
        pl.kernel,
        mesh=mesh,
        out_type=jax.ShapeDtypeStruct((T, D), jnp.float32),
        scratch_types=[
            pltpu.VMEM((2, KCODES, 128), jnp.int32),
            pltpu.VMEM((4, SUB * KCODES, D), jnp.float32),
            pltpu.VMEM((OUTER, D), jnp.float32),
            pltpu.SemaphoreType.DMA,
            pltpu.SemaphoreType.DMA,
            pltpu.SemaphoreType.DMA,
            pltpu.SemaphoreType.DMA,
            pltpu.SemaphoreType.DMA,
            pltpu.SemaphoreType.DMA,
        ],
    )
    def sc_gather(p_hbm, codes_hbm, out_hbm, idx_v, rows_v, acc_v,
                  sem0, sem1, sem2, sem3, idx_sem, out_sem):
        wid = lax.axis_index("s") * NC + lax.axis_index("c")
        sems = (sem0, sem1, sem2, sem3)
        base0 = pl.multiple_of(wid * TPW, OUTER)

        def idx_row_of(jo):
            return pl.multiple_of((base0 + jo * OUTER) // (128 // KCODES),
                                  KCODES)

        def issue(jb, sub, buf):
            pltpu.async_copy(
                p_hbm.at[idx_v.at[jb, sub]], rows_v.at[buf], sems[buf])

        def wait_rows(buf):
            pltpu.make_async_copy(
                p_hbm.at[idx_v.at[0, 0]], rows_v.at[buf], sems[buf]).wait()

        def idx_prefetch(jo, jb):
            return pltpu.make_async_copy(
                codes_hbm.at[pl.ds(idx_row_of(jo), KCODES)],
                idx_v.at[jb], idx_sem)

        def out_copy(base_tok):
            return pltpu.make_async_copy(
                acc_v, out_hbm.at[pl.ds(base_tok, OUTER)], out_sem)

        pltpu.sync_copy(codes_hbm.at[pl.ds(idx_row_of(0), KCODES)],
                        idx_v.at[0])
        for s in range(4):
            issue(0, s, s)
        if NOUT > 1:
            idx_prefetch(1, 1).start()

        def chunk_body(j, carry):
            base_tok = pl.multiple_of(base0 + j * OUTER, OUTER)
            jb = lax.rem(j, 2)
            jb_next = lax.rem(j + 1, 2)

            @pl.when(j > 0)
            def _():
                out_copy(base_tok - OUTER).wait()

            for sub in range(NSUB):
                buf = sub % 4
                wait_rows(buf)

                def pair_body(p, carry2, _sub=sub, _buf=buf):
                    for t in range(2):
                        i = p * 2 + t
                        r = i * KCODES
                        row = _sub * SUB + i
                        for jj in range(D // LANES):
                            sl = pl.ds(jj * LANES, LANES)
                            v = [rows_v[_buf, r + k, sl]
                                 for k in range(KCODES)]
                            s = (((v[0] + v[1]) + (v[2] + v[3]))
                                 + ((v[4] + v[5]) + (v[6] + v[7])))
                            acc_v[row, sl] = s
                    return carry2

                lax.fori_loop(0, SUB // 2, pair_body, 0)

                if sub < 4:
                    issue(jb, sub + 4, buf)
                else:
                    if sub == 4:
                        @pl.when(j + 1 < NOUT)
                        def _():
                            idx_prefetch(0, 0).wait()
                            issue(jb_next, 0, buf)
                    else:
                        @pl.when(j + 1 < NOUT)
                        def _():
                            issue(jb_next, sub - 4, buf)
                    if sub == 7:
                        @pl.when(j + 2 < NOUT)
                        def _():
                            idx_prefetch(j + 2, jb).start()
            out_copy(base_tok).start()
            return carry

        lax.fori_loop(0, NOUT, chunk_body, 0)
        out_copy(base0 + (NOUT - 1) * OUTER).wait()

    return sc_gather


def kernel(codes, codebook, W, b, ln_gamma, ln_beta):
    Bb, Nn, Kk = codes.shape
    T = Bb * Nn
    codes_flat = codes.astype(jnp.int32).reshape(T * Kk // 128, 128)
    cb_pad = jnp.pad(codebook, ((0, 7), (0, 0)))
    P = pl.pallas_call(
        _proj_body,
        out_shape=jax.ShapeDtypeStruct((cb_pad.shape[0], D), jnp.float32),
    )(cb_pad, W, b.reshape(1, D))
    NSLAB = 4
    TS = T // NSLAB
    TB = 8192
    sc_call = _make_sc_gather(TS)
    g2 = ln_gamma.reshape(1, D)
    b2 = ln_beta.reshape(1, D)
    rows_per_slab = TS * Kk // 128
    outs = []
    for sl in range(NSLAB):
        sums = sc_call(
            P, lax.slice_in_dim(codes_flat, sl * rows_per_slab,
                                (sl + 1) * rows_per_slab))
        outs.append(pl.pallas_call(
            _ln_body,
            grid=(TS // TB,),
            in_specs=[
                pl.BlockSpec((TB, D), lambda i: (i, 0)),
                pl.BlockSpec((1, D), lambda i: (0, 0)),
                pl.BlockSpec((1, D), lambda i: (0, 0)),
            ],
            out_specs=pl.BlockSpec((TB, D), lambda i: (i, 0)),
            out_shape=jax.ShapeDtypeStruct((TS, D), jnp.float32),
        )(sums, g2, b2))
    out = jnp.concatenate(outs, axis=0)
    return out.reshape(Bb, Nn, D)


# speedup vs baseline: 1.1033x; 1.1033x over previous
import functools

import jax
import jax.numpy as jnp
from jax import lax
from jax.experimental import pallas as pl
from jax.experimental.pallas import tpu as pltpu
from jax.experimental.pallas import tpu_sc as plsc

LN_EPS = 1e-5
LANES = 16
NC = 2
NS = 16
NW = NC * NS
KCODES = 8
D = 128
SUB = 16
OUTER = 128
NSUB = OUTER // SUB


def _proj_body(cb_ref, w_ref, b_ref, o_ref):
    o_ref[...] = (
        jnp.dot(cb_ref[...], w_ref[...], preferred_element_type=jnp.float32)
        + b_ref[...]
    ) * 0.125


def _ln_body(x_ref, g_ref, bb_ref, o_ref):
    x = x_ref[...]
    mu = jnp.mean(x, axis=-1, keepdims=True)
    xc = x - mu
    var = jnp.mean(xc * xc, axis=-1, keepdims=True)
    o_ref[...] = xc * lax.rsqrt(var + LN_EPS) * g_ref[...] + bb_ref[...]


@functools.lru_cache(maxsize=None)
def _make_sc_gather(T):
    TPW = T // NW
    NOUT = TPW // OUTER
    mesh = plsc.VectorSubcoreMesh(core_axis_name="c", subcore_axis_name="s")

    @functools.partial(
        pl.kernel,
        mesh=mesh,
        out_type=jax.ShapeDtypeStruct((T, D), jnp.float32),
        scratch_types=[
            pltpu.VMEM_SHARED((1032, D), jnp.float32),
            pltpu.VMEM((2, KCODES, 128), jnp.int32),
            pltpu.VMEM((4, SUB * KCODES, D), jnp.float32),
            pltpu.VMEM((OUTER, D), jnp.float32),
            pltpu.SemaphoreType.DMA,
            pltpu.SemaphoreType.DMA,
            pltpu.SemaphoreType.DMA,
            pltpu.SemaphoreType.DMA,
            pltpu.SemaphoreType.DMA,
            pltpu.SemaphoreType.DMA,
        ],
    )
    def sc_gather(p_hbm, codes_hbm, out_hbm, p_sh, idx_v, rows_v, acc_v,
                  sem0, sem1, sem2, sem3, idx_sem, out_sem):
        sid = lax.axis_index("s")
        wid = sid * NC + lax.axis_index("c")
        sems = (sem0, sem1, sem2, sem3)
        base0 = pl.multiple_of(wid * TPW, OUTER)

        def idx_row_of(jo):
            return pl.multiple_of((base0 + jo * OUTER) // (128 // KCODES),
                                  KCODES)

        def issue(jb, sub, buf):
            pltpu.async_copy(
                p_sh.at[idx_v.at[jb, sub]], rows_v.at[buf], sems[buf])

        def wait_rows(buf):
            pltpu.make_async_copy(
                p_sh.at[idx_v.at[0, 0]], rows_v.at[buf], sems[buf]).wait()

        def idx_prefetch(jo, jb):
            return pltpu.make_async_copy(
                codes_hbm.at[pl.ds(idx_row_of(jo), KCODES)],
                idx_v.at[jb], idx_sem)

        def out_copy(base_tok):
            return pltpu.make_async_copy(
                acc_v, out_hbm.at[pl.ds(base_tok, OUTER)], out_sem)

        @pl.when(sid == 0)
        def _():
            pltpu.sync_copy(p_hbm, p_sh)
        plsc.subcore_barrier()
        pltpu.sync_copy(codes_hbm.at[pl.ds(idx_row_of(0), KCODES)],
                        idx_v.at[0])
        for s in range(4):
            issue(0, s, s)
        if NOUT > 1:
            idx_prefetch(1, 1).start()

        def chunk_body(j, carry):
            base_tok = pl.multiple_of(base0 + j * OUTER, OUTER)
            jb = lax.rem(j, 2)
            jb_next = lax.rem(j + 1, 2)

            @pl.when(j > 0)
            def _():
                out_copy(base_tok - OUTER).wait()

            for sub in range(NSUB):
                buf = sub % 4
                wait_rows(buf)

                def pair_body(p, carry2, _sub=sub, _buf=buf):
                    for t in range(2):
                        i = p * 2 + t
                        r = i * KCODES
                        row = _sub * SUB + i
                        for jj in range(D // LANES):
                            sl = pl.ds(jj * LANES, LANES)
                            v = [rows_v[_buf, r + k, sl]
                                 for k in range(KCODES)]
                            s = (((v[0] + v[1]) + (v[2] + v[3]))
                                 + ((v[4] + v[5]) + (v[6] + v[7])))
                            acc_v[row, sl] = s
                    return carry2

                lax.fori_loop(0, SUB // 2, pair_body, 0)

                if sub < 4:
                    issue(jb, sub + 4, buf)
                else:
                    if sub == 4:
                        @pl.when(j + 1 < NOUT)
                        def _():
                            idx_prefetch(0, 0).wait()
                            issue(jb_next, 0, buf)
                    else:
                        @pl.when(j + 1 < NOUT)
                        def _():
                            issue(jb_next, sub - 4, buf)
                    if sub == 7:
                        @pl.when(j + 2 < NOUT)
                        def _():
                            idx_prefetch(j + 2, jb).start()
            out_copy(base_tok).start()
            return carry

        lax.fori_loop(0, NOUT, chunk_body, 0)
        out_copy(base0 + (NOUT - 1) * OUTER).wait()

    return sc_gather


def kernel(codes, codebook, W, b, ln_gamma, ln_beta):
    Bb, Nn, Kk = codes.shape
    T = Bb * Nn
    codes_flat = codes.astype(jnp.int32).reshape(T * Kk // 128, 128)
    cb_pad = jnp.pad(codebook, ((0, 7), (0, 0)))
    P = pl.pallas_call(
        _proj_body,
        out_shape=jax.ShapeDtypeStruct((cb_pad.shape[0], D), jnp.float32),
    )(cb_pad, W, b.reshape(1, D))
    sums = _make_sc_gather(T)(P, codes_flat)
    TB = 8192
    out = pl.pallas_call(
        _ln_body,
        grid=(T // TB,),
        in_specs=[
            pl.BlockSpec((TB, D), lambda i: (i, 0)),
            pl.BlockSpec((1, D), lambda i: (0, 0)),
            pl.BlockSpec((1, D), lambda i: (0, 0)),
        ],
        out_specs=pl.BlockSpec((TB, D), lambda i: (i, 0)),
        out_shape=jax.ShapeDtypeStruct((T, D), jnp.float32),
    )(sums, ln_gamma.reshape(1, D), ln_beta.reshape(1, D))
    return out.reshape(Bb, Nn, D)


# speedup vs baseline: 1.1242x; 1.0189x over previous
import functools

import jax
import jax.numpy as jnp
from jax import lax
from jax.experimental import pallas as pl
from jax.experimental.pallas import tpu as pltpu
from jax.experimental.pallas import tpu_sc as plsc

LN_EPS = 1e-5
LANES = 16
NC = 2
NS = 16
NW = NC * NS
KCODES = 8
D = 128
SUB = 16
OUTER = 128
NSUB = OUTER // SUB


def _proj_body(cb_ref, w_ref, b_ref, o_ref):
    o_ref[...] = (
        jnp.dot(cb_ref[...], w_ref[...], preferred_element_type=jnp.float32)
        + b_ref[...]
    ) * 0.125


def _ln_body(x_ref, g_ref, bb_ref, o_ref):
    x = x_ref[...]
    mu = jnp.mean(x, axis=-1, keepdims=True)
    xc = x - mu
    var = jnp.mean(xc * xc, axis=-1, keepdims=True)
    o_ref[...] = xc * lax.rsqrt(var + LN_EPS) * g_ref[...] + bb_ref[...]


CBV = 1032
TTB = 512


def _onehot_ln_body(codes_ref, pb_ref, g_ref, bb_ref, o_ref):
    codes = codes_ref[...]
    iota2 = lax.broadcasted_iota(jnp.int32, (TTB, CBV), 1)
    counts = None
    for k in range(KCODES):
        m = (iota2 == codes[:, k][:, None]).astype(jnp.bfloat16)
        counts = m if counts is None else counts + m
    x = jnp.dot(counts, pb_ref[...], preferred_element_type=jnp.float32)
    mu = jnp.mean(x, axis=-1, keepdims=True)
    xc = x - mu
    var = jnp.mean(xc * xc, axis=-1, keepdims=True)
    o_ref[...] = xc * lax.rsqrt(var + LN_EPS) * g_ref[...] + bb_ref[...]


@functools.lru_cache(maxsize=None)
def _make_sc_gather(T):
    TPW = T // NW
    NOUT = TPW // OUTER
    mesh = plsc.VectorSubcoreMesh(core_axis_name="c", subcore_axis_name="s")

    @functools.partial(
        pl.kernel,
        mesh=mesh,
        out_type=jax.ShapeDtypeStruct((T, D), jnp.float32),
        scratch_types=[
            pltpu.VMEM_SHARED((1032, D), jnp.float32),
            pltpu.VMEM((2, KCODES, 128), jnp.int32),
            pltpu.VMEM((4, SUB * KCODES, D), jnp.float32),
            pltpu.VMEM((OUTER, D), jnp.float32),
            pltpu.SemaphoreType.DMA,
            pltpu.SemaphoreType.DMA,
            pltpu.SemaphoreType.DMA,
            pltpu.SemaphoreType.DMA,
            pltpu.SemaphoreType.DMA,
            pltpu.SemaphoreType.DMA,
        ],
    )
    def sc_gather(p_hbm, codes_hbm, out_hbm, p_sh, idx_v, rows_v, acc_v,
                  sem0, sem1, sem2, sem3, idx_sem, out_sem):
        sid = lax.axis_index("s")
        wid = sid * NC + lax.axis_index("c")
        sems = (sem0, sem1, sem2, sem3)
        base0 = pl.multiple_of(wid * TPW, OUTER)

        def idx_row_of(jo):
            return pl.multiple_of((base0 + jo * OUTER) // (128 // KCODES),
                                  KCODES)

        def issue(jb, sub, buf):
            pltpu.async_copy(
                p_sh.at[idx_v.at[jb, sub]], rows_v.at[buf], sems[buf])

        def wait_rows(buf):
            pltpu.make_async_copy(
                p_sh.at[idx_v.at[0, 0]], rows_v.at[buf], sems[buf]).wait()

        def idx_prefetch(jo, jb):
            return pltpu.make_async_copy(
                codes_hbm.at[pl.ds(idx_row_of(jo), KCODES)],
                idx_v.at[jb], idx_sem)

        def out_copy(base_tok):
            return pltpu.make_async_copy(
                acc_v, out_hbm.at[pl.ds(base_tok, OUTER)], out_sem)

        @pl.when(sid == 0)
        def _():
            pltpu.sync_copy(p_hbm, p_sh)
        plsc.subcore_barrier()
        pltpu.sync_copy(codes_hbm.at[pl.ds(idx_row_of(0), KCODES)],
                        idx_v.at[0])
        for s in range(4):
            issue(0, s, s)
        if NOUT > 1:
            idx_prefetch(1, 1).start()

        def chunk_body(j, carry):
            base_tok = pl.multiple_of(base0 + j * OUTER, OUTER)
            jb = lax.rem(j, 2)
            jb_next = lax.rem(j + 1, 2)

            @pl.when(j > 0)
            def _():
                out_copy(base_tok - OUTER).wait()

            for sub in range(NSUB):
                buf = sub % 4
                wait_rows(buf)

                def pair_body(p, carry2, _sub=sub, _buf=buf):
                    for t in range(2):
                        i = p * 2 + t
                        r = i * KCODES
                        row = _sub * SUB + i
                        for jj in range(D // LANES):
                            sl = pl.ds(jj * LANES, LANES)
                            v = [rows_v[_buf, r + k, sl]
                                 for k in range(KCODES)]
                            s = (((v[0] + v[1]) + (v[2] + v[3]))
                                 + ((v[4] + v[5]) + (v[6] + v[7])))
                            acc_v[row, sl] = s
                    return carry2

                lax.fori_loop(0, SUB // 2, pair_body, 0)

                if sub < 4:
                    issue(jb, sub + 4, buf)
                else:
                    if sub == 4:
                        @pl.when(j + 1 < NOUT)
                        def _():
                            idx_prefetch(0, 0).wait()
                            issue(jb_next, 0, buf)
                    else:
                        @pl.when(j + 1 < NOUT)
                        def _():
                            issue(jb_next, sub - 4, buf)
                    if sub == 7:
                        @pl.when(j + 2 < NOUT)
                        def _():
                            idx_prefetch(j + 2, jb).start()
            out_copy(base_tok).start()
            return carry

        lax.fori_loop(0, NOUT, chunk_body, 0)
        out_copy(base0 + (NOUT - 1) * OUTER).wait()

    return sc_gather


def kernel(codes, codebook, W, b, ln_gamma, ln_beta):
    Bb, Nn, Kk = codes.shape
    T = Bb * Nn
    codes_flat = codes.astype(jnp.int32).reshape(T * Kk // 128, 128)
    cb_pad = jnp.pad(codebook, ((0, 7), (0, 0)))
    P = pl.pallas_call(
        _proj_body,
        out_shape=jax.ShapeDtypeStruct((cb_pad.shape[0], D), jnp.float32),
    )(cb_pad, W, b.reshape(1, D))
    g2 = ln_gamma.reshape(1, D)
    b2 = ln_beta.reshape(1, D)
    TSC = 40 * 8192
    TTC = T - TSC
    sums = _make_sc_gather(TSC)(P, codes_flat[: TSC * Kk // 128])
    out_tc = pl.pallas_call(
        _onehot_ln_body,
        grid=(TTC // TTB,),
        in_specs=[
            pl.BlockSpec((TTB, KCODES), lambda i: (i, 0)),
            pl.BlockSpec((CBV, D), lambda i: (0, 0)),
            pl.BlockSpec((1, D), lambda i: (0, 0)),
            pl.BlockSpec((1, D), lambda i: (0, 0)),
        ],
        out_specs=pl.BlockSpec((TTB, D), lambda i: (i, 0)),
        out_shape=jax.ShapeDtypeStruct((TTC, D), jnp.float32),
    )(codes.reshape(T, Kk)[TSC:].astype(jnp.int32), P.astype(jnp.bfloat16),
      g2, b2)
    TB = 8192
    out_sc = pl.pallas_call(
        _ln_body,
        grid=(TSC // TB,),
        in_specs=[
            pl.BlockSpec((TB, D), lambda i: (i, 0)),
            pl.BlockSpec((1, D), lambda i: (0, 0)),
            pl.BlockSpec((1, D), lambda i: (0, 0)),
        ],
        out_specs=pl.BlockSpec((TB, D), lambda i: (i, 0)),
        out_shape=jax.ShapeDtypeStruct((TSC, D), jnp.float32),
    )(sums, g2, b2)
    out = jnp.concatenate([out_sc, out_tc], axis=0)
    return out.reshape(Bb, Nn, D)
